# baseline (speedup 1.0000x reference)
import jax
import jax.numpy as jnp
from jax import lax
from jax.experimental import pallas as pl
from jax.experimental.pallas import tpu as pltpu
from jax.experimental.pallas import tpu_sc as plsc

_B, _S, _D = 4, 8192, 1024
_NC, _NS = 2, 16
_NW = _NC * _NS
_SPW = _S // _NW
_CH = 8
_NCHUNK = _SPW // _CH
_CHW = _CH * _D


def _sc_body(x_hbm, emb_hbm, out_hbm,
             xb0, xb1, xb2, xb3, xb4, xb5, xb6, xb7, eb0, eb1,
             xsem, osem, esem):
    xb = (xb0, xb1, xb2, xb3, xb4, xb5, xb6, xb7)
    eb = (eb0, eb1)
    wid = lax.axis_index("s") * _NC + lax.axis_index("c")
    s0 = wid * _SPW

    def eoff(c):
        return (s0 + c * _CH) * _D

    def xoff(c, k):
        return (k * _S + s0 + c * _CH) * _D

    def ein_desc(c, par):
        return pltpu.make_async_copy(
            emb_hbm.at[pl.ds(eoff(c), _CHW)], eb[par], esem.at[par])

    def xin_desc(c, k, par):
        r = par * 4 + k
        return pltpu.make_async_copy(
            x_hbm.at[pl.ds(xoff(c, k), _CHW)], xb[r], xsem.at[r])

    def out_desc(c, k, par):
        r = par * 4 + k
        return pltpu.make_async_copy(
            xb[r], out_hbm.at[pl.ds(xoff(c, k), _CHW)], osem.at[r])

    def chunk(c, par, issue_ein_next=True, wait_out_prev=True,
              issue_xin_next=True):
        if issue_ein_next:
            ein_desc(c + 1, 1 - par).start()
        for k in range(_B):
            xin_desc(c, k, par).wait()
        ein_desc(c, par).wait()
        bufs = xb[par * 4:par * 4 + _B]
        e = eb[par]

        @plsc.parallel_loop(0, _CHW, 16, unroll=4)
        def add_body(i):
            ev = e[pl.ds(i, 16)]
            for k in range(_B):
                bufs[k][pl.ds(i, 16)] = bufs[k][pl.ds(i, 16)] + ev

        for k in range(_B):
            out_desc(c, k, par).start()
        for k in range(_B):
            if wait_out_prev:
                out_desc(c - 1, k, 1 - par).wait()
            if issue_xin_next:
                xin_desc(c + 1, k, 1 - par).start()

    ein_desc(0, 0).start()
    for k in range(_B):
        xin_desc(0, k, 0).start()
    chunk(0, 0, wait_out_prev=False)

    def pair_body(p, carry):
        c1 = 1 + 2 * p
        chunk(c1, 1)
        chunk(c1 + 1, 0)
        return carry

    lax.fori_loop(0, (_NCHUNK - 2) // 2, pair_body, 0)

    chunk(_NCHUNK - 1, 1, issue_ein_next=False, issue_xin_next=False)
    for k in range(_B):
        out_desc(_NCHUNK - 1, k, 1).wait()


def kernel(x, emb_table):
    B, S, D = x.shape
    mesh = plsc.VectorSubcoreMesh(core_axis_name="c", subcore_axis_name="s")
    out_flat = pl.kernel(
        _sc_body,
        out_type=jax.ShapeDtypeStruct((B * S * D,), jnp.float32),
        mesh=mesh,
        scratch_types=(
            [pltpu.VMEM((_CHW,), jnp.float32) for _ in range(8)]
            + [pltpu.VMEM((_CHW,), jnp.float32) for _ in range(2)]
            + [pltpu.SemaphoreType.DMA((8,)),
               pltpu.SemaphoreType.DMA((8,)),
               pltpu.SemaphoreType.DMA((2,))]
        ),
    )(x.reshape(-1), emb_table.reshape(-1))
    return out_flat.reshape(B, S, D)


# speedup vs baseline: 3.1549x; 3.1549x over previous
import jax
import jax.numpy as jnp
from jax import lax
from jax.experimental import pallas as pl
from jax.experimental.pallas import tpu as pltpu
from jax.experimental.pallas import tpu_sc as plsc

_B, _S, _D = 4, 8192, 1024
_NC, _NS = 2, 16
_NW = _NC * _NS
_SPW = _S // _NW
_CH = 8
_NCHUNK = _SPW // _CH


def _sc_body(x_hbm, emb_hbm, out_hbm,
             xb0, xb1, xb2, xb3, xb4, xb5, xb6, xb7, eb0, eb1,
             xsem, osem, esem):
    xb = (xb0, xb1, xb2, xb3, xb4, xb5, xb6, xb7)
    eb = (eb0, eb1)
    wid = lax.axis_index("s") * _NC + lax.axis_index("c")
    s0 = wid * _SPW

    def row(c):
        return s0 + c * _CH

    def ein_desc(c, par):
        return pltpu.make_async_copy(
            emb_hbm.at[pl.ds(row(c), _CH)], eb[par], esem.at[par])

    def xin_desc(c, k, par):
        r = par * 4 + k
        return pltpu.make_async_copy(
            x_hbm.at[k, pl.ds(row(c), _CH)], xb[r], xsem.at[r])

    def out_desc(c, k, par):
        r = par * 4 + k
        return pltpu.make_async_copy(
            xb[r], out_hbm.at[k, pl.ds(row(c), _CH)], osem.at[r])

    def chunk(c, par, issue_ein_next=True, wait_out_prev=True,
              issue_xin_next=True):
        if issue_ein_next:
            ein_desc(c + 1, 1 - par).start()
        ein_desc(c, par).wait()
        e = eb[par]
        for k in range(_B):
            xin_desc(c, k, par).wait()
            buf = xb[par * 4 + k]
            for r in range(_CH):

                @plsc.parallel_loop(0, _D, 16, unroll=8)
                def add_body(i):
                    buf[r, pl.ds(i, 16)] = buf[r, pl.ds(i, 16)] + e[r, pl.ds(i, 16)]

            out_desc(c, k, par).start()
            if wait_out_prev:
                out_desc(c - 1, k, 1 - par).wait()
            if issue_xin_next:
                xin_desc(c + 1, k, 1 - par).start()

    ein_desc(0, 0).start()
    for k in range(_B):
        xin_desc(0, k, 0).start()
    chunk(0, 0, wait_out_prev=False)

    def pair_body(p, carry):
        c1 = 1 + 2 * p
        chunk(c1, 1)
        chunk(c1 + 1, 0)
        return carry

    lax.fori_loop(0, (_NCHUNK - 2) // 2, pair_body, 0)

    chunk(_NCHUNK - 1, 1, issue_ein_next=False, issue_xin_next=False)
    for k in range(_B):
        out_desc(_NCHUNK - 1, k, 1).wait()


def kernel(x, emb_table):
    B, S, D = x.shape
    mesh = plsc.VectorSubcoreMesh(core_axis_name="c", subcore_axis_name="s")
    return pl.kernel(
        _sc_body,
        out_type=jax.ShapeDtypeStruct((B, S, D), jnp.float32),
        mesh=mesh,
        scratch_types=(
            [pltpu.VMEM((_CH, _D), jnp.float32) for _ in range(10)]
            + [pltpu.SemaphoreType.DMA((8,)),
               pltpu.SemaphoreType.DMA((8,)),
               pltpu.SemaphoreType.DMA((2,))]
        ),
    )(x, emb_table)


# speedup vs baseline: 3.2006x; 1.0145x over previous
import jax
import jax.numpy as jnp
from jax import lax
from jax.experimental import pallas as pl
from jax.experimental.pallas import tpu as pltpu
from jax.experimental.pallas import tpu_sc as plsc

_B, _S, _D = 4, 8192, 1024
_NC, _NS = 2, 16
_NW = _NC * _NS
_SPW = _S // _NW
_CH = 8
_NCHUNK = _SPW // _CH


def _sc_body(x_hbm, emb_hbm, out_hbm,
             xb0, xb1, xb2, xb3, xb4, xb5, xb6, xb7, eb0, eb1,
             xsem, osem, esem):
    xb = (xb0, xb1, xb2, xb3, xb4, xb5, xb6, xb7)
    eb = (eb0, eb1)
    wid = lax.axis_index("s") * _NC + lax.axis_index("c")
    s0 = wid * _SPW

    def row(c):
        return s0 + c * _CH

    def ein_desc(c, par):
        return pltpu.make_async_copy(
            emb_hbm.at[pl.ds(row(c), _CH)], eb[par], esem.at[par])

    def xin_desc(c, k, par):
        r = par * 4 + k
        return pltpu.make_async_copy(
            x_hbm.at[k, pl.ds(row(c), _CH)], xb[r], xsem.at[r])

    def out_desc(c, k, par):
        r = par * 4 + k
        return pltpu.make_async_copy(
            xb[r], out_hbm.at[k, pl.ds(row(c), _CH)], osem.at[r])

    def chunk(c, par, issue_ein_next=True, wait_out_prev=True,
              issue_xin_next=True):
        if issue_ein_next:
            ein_desc(c + 1, 1 - par).start()
        ein_desc(c, par).wait()
        for k in range(_B):
            xin_desc(c, k, par).wait()
        e = eb[par]
        bufs = xb[par * 4:par * 4 + _B]
        for r in range(_CH):

            @plsc.parallel_loop(0, _D, 16, unroll=4)
            def add_body(i):
                ev = e[r, pl.ds(i, 16)]
                for k in range(_B):
                    bufs[k][r, pl.ds(i, 16)] = bufs[k][r, pl.ds(i, 16)] + ev

        for k in range(_B):
            out_desc(c, k, par).start()
        for k in range(_B):
            if wait_out_prev:
                out_desc(c - 1, k, 1 - par).wait()
            if issue_xin_next:
                xin_desc(c + 1, k, 1 - par).start()

    ein_desc(0, 0).start()
    for k in range(_B):
        xin_desc(0, k, 0).start()
    chunk(0, 0, wait_out_prev=False)

    def pair_body(p, carry):
        c1 = 1 + 2 * p
        chunk(c1, 1)
        chunk(c1 + 1, 0)
        return carry

    lax.fori_loop(0, (_NCHUNK - 2) // 2, pair_body, 0)

    chunk(_NCHUNK - 1, 1, issue_ein_next=False, issue_xin_next=False)
    for k in range(_B):
        out_desc(_NCHUNK - 1, k, 1).wait()


def kernel(x, emb_table):
    B, S, D = x.shape
    mesh = plsc.VectorSubcoreMesh(core_axis_name="c", subcore_axis_name="s")
    return pl.kernel(
        _sc_body,
        out_type=jax.ShapeDtypeStruct((B, S, D), jnp.float32),
        mesh=mesh,
        scratch_types=(
            [pltpu.VMEM((_CH, _D), jnp.float32) for _ in range(10)]
            + [pltpu.SemaphoreType.DMA((8,)),
               pltpu.SemaphoreType.DMA((8,)),
               pltpu.SemaphoreType.DMA((2,))]
        ),
    )(x, emb_table)


# speedup vs baseline: 3.4738x; 1.0854x over previous
import jax
import jax.numpy as jnp
from jax import lax
from jax.experimental import pallas as pl
from jax.experimental.pallas import tpu as pltpu
from jax.experimental.pallas import tpu_sc as plsc

_B, _S, _D = 4, 8192, 1024
_NC, _NS = 2, 16
_NW = _NC * _NS
_SPW = _S // _NW
_CH = 8
_NCHUNK = _SPW // _CH
_NPAR = 3


def _sc_body(x_hbm, emb_hbm, out_hbm,
             xb0, xb1, xb2, xb3, xb4, xb5, xb6, xb7, xb8, xb9, xb10, xb11,
             eb0, eb1, eb2, xsem, osem, esem):
    xb = (xb0, xb1, xb2, xb3, xb4, xb5, xb6, xb7, xb8, xb9, xb10, xb11)
    eb = (eb0, eb1, eb2)
    wid = lax.axis_index("s") * _NC + lax.axis_index("c")
    s0 = wid * _SPW

    def row(c):
        return s0 + c * _CH

    def ein_desc(c, par):
        return pltpu.make_async_copy(
            emb_hbm.at[pl.ds(row(c), _CH)], eb[par], esem.at[par])

    def xin_desc(c, k, par):
        r = par * 4 + k
        return pltpu.make_async_copy(
            x_hbm.at[k, pl.ds(row(c), _CH)], xb[r], xsem.at[r])

    def out_desc(c, k, par):
        r = par * 4 + k
        return pltpu.make_async_copy(
            xb[r], out_hbm.at[k, pl.ds(row(c), _CH)], osem.at[r])

    def chunk(c, par, issue_next=True, wait_out_prev=True):
        par2 = (par + 2) % _NPAR
        par1 = (par + _NPAR - 1) % _NPAR
        if issue_next:
            ein_desc(c + 2, par2).start()
        ein_desc(c, par).wait()
        e = eb[par]
        for k in range(_B):
            xin_desc(c, k, par).wait()
            if wait_out_prev:
                out_desc(c - 1, k, par1).wait()
            if issue_next:
                xin_desc(c + 2, k, par2).start()
            buf = xb[par * 4 + k]

            @plsc.parallel_loop(0, _CH * _D, 16, unroll=8)
            def add_body(i):
                r = i >> 10
                col = pl.multiple_of(i & (_D - 1), 16)
                buf[r, pl.ds(col, 16)] = buf[r, pl.ds(col, 16)] + e[r, pl.ds(col, 16)]

            out_desc(c, k, par).start()

    for c in range(2):
        ein_desc(c, c).start()
        for k in range(_B):
            xin_desc(c, k, c).start()
    chunk(0, 0, wait_out_prev=False)
    chunk(1, 1)

    def triple_body(p, carry):
        c1 = 2 + 3 * p
        chunk(c1, 2)
        chunk(c1 + 1, 0)
        chunk(c1 + 2, 1)
        return carry

    lax.fori_loop(0, (_NCHUNK - 5) // 3, triple_body, 0)

    chunk(_NCHUNK - 3, 2)
    chunk(_NCHUNK - 2, 0, issue_next=False)
    chunk(_NCHUNK - 1, 1, issue_next=False)
    for k in range(_B):
        out_desc(_NCHUNK - 1, k, 1).wait()


def kernel(x, emb_table):
    B, S, D = x.shape
    mesh = plsc.VectorSubcoreMesh(core_axis_name="c", subcore_axis_name="s")
    return pl.kernel(
        _sc_body,
        out_type=jax.ShapeDtypeStruct((B, S, D), jnp.float32),
        mesh=mesh,
        scratch_types=(
            [pltpu.VMEM((_CH, _D), jnp.float32) for _ in range(15)]
            + [pltpu.SemaphoreType.DMA((12,)),
               pltpu.SemaphoreType.DMA((12,)),
               pltpu.SemaphoreType.DMA((3,))]
        ),
    )(x, emb_table)
